# initial kernel scaffold (unmeasured)
import jax
import jax.numpy as jnp
from jax import lax
from jax.experimental import pallas as pl
from jax.experimental.pallas import tpu as pltpu


def kernel(
    x,
):
    def body(*refs):
        pass

    out_shape = jax.ShapeDtypeStruct(..., jnp.float32)
    return pl.pallas_call(body, out_shape=out_shape)(...)


# baseline (device time: 228813 ns/iter reference)
import jax
import jax.numpy as jnp
from jax import lax
from jax.experimental import pallas as pl
from jax.experimental.pallas import tpu as pltpu


def kernel(x):
    x = x.astype(jnp.bfloat16)
    m, n = x.shape

    def body(x_ref, out_ref, send_sem, recv_sem):
        my_x = lax.axis_index("x")
        my_y = lax.axis_index("y")
        my_z = lax.axis_index("z")
        peer = (my_x, 1 - my_y, my_z)

        bar = pltpu.get_barrier_semaphore()
        pl.semaphore_signal(
            bar, inc=1, device_id=peer, device_id_type=pl.DeviceIdType.MESH
        )
        pl.semaphore_wait(bar, 1)

        my_off = my_y * m
        rdma = pltpu.make_async_remote_copy(
            src_ref=x_ref,
            dst_ref=out_ref.at[pl.ds(my_off, m), :],
            send_sem=send_sem,
            recv_sem=recv_sem,
            device_id=peer,
            device_id_type=pl.DeviceIdType.MESH,
        )
        rdma.start()
        out_ref[pl.ds(my_off, m), :] = x_ref[:, :]
        rdma.wait()

    return pl.pallas_call(
        body,
        out_shape=jax.ShapeDtypeStruct((2 * m, n), jnp.bfloat16),
        in_specs=[pl.BlockSpec(memory_space=pltpu.VMEM)],
        out_specs=pl.BlockSpec(memory_space=pltpu.VMEM),
        scratch_shapes=[
            pltpu.SemaphoreType.DMA,
            pltpu.SemaphoreType.DMA,
        ],
        compiler_params=pltpu.CompilerParams(collective_id=0),
    )(x)


# device time: 148818 ns/iter; 1.5375x vs baseline; 1.5375x over previous
import jax
import jax.numpy as jnp
from jax import lax
from jax.experimental import pallas as pl
from jax.experimental.pallas import tpu as pltpu

C = 16


def kernel(x):
    x = x.astype(jnp.bfloat16)
    m, n = x.shape
    half = m // 2
    cs = half // C

    def body(x_ref, out_ref, y_send, y_recv, x_send, x_recv, loc_sem):
        my_x = lax.axis_index("x")
        my_y = lax.axis_index("y")
        my_z = lax.axis_index("z")
        y_peer = (my_x, 1 - my_y, my_z)
        x_peer = (1 - my_x, my_y, my_z)

        bar = pltpu.get_barrier_semaphore()
        for p in (y_peer, x_peer):
            pl.semaphore_signal(
                bar, inc=1, device_id=p, device_id_type=pl.DeviceIdType.MESH
            )
        pl.semaphore_wait(bar, 2)

        half_off = my_x * half
        mine_off = my_y * m
        rem_off = (1 - my_y) * m

        loc = pltpu.make_async_copy(
            x_ref, out_ref.at[pl.ds(mine_off, m), :], loc_sem
        )
        loc.start()

        y_rdmas = []
        for c in range(C):
            src_o = half_off + c * cs
            rdma = pltpu.make_async_remote_copy(
                src_ref=x_ref.at[pl.ds(src_o, cs), :],
                dst_ref=out_ref.at[pl.ds(mine_off + src_o, cs), :],
                send_sem=y_send.at[c],
                recv_sem=y_recv.at[c],
                device_id=y_peer,
                device_id_type=pl.DeviceIdType.MESH,
            )
            rdma.start()
            y_rdmas.append(rdma)

        x_rdmas = []
        for c in range(C):
            y_rdmas[c].wait_recv()
            f_off = rem_off + half_off + c * cs
            fwd = pltpu.make_async_remote_copy(
                src_ref=out_ref.at[pl.ds(f_off, cs), :],
                dst_ref=out_ref.at[pl.ds(f_off, cs), :],
                send_sem=x_send.at[c],
                recv_sem=x_recv.at[c],
                device_id=x_peer,
                device_id_type=pl.DeviceIdType.MESH,
            )
            fwd.start()
            x_rdmas.append(fwd)

        for c in range(C):
            x_rdmas[c].wait_recv()
            y_rdmas[c].wait_send()
            x_rdmas[c].wait_send()
        loc.wait()

    return pl.pallas_call(
        body,
        out_shape=jax.ShapeDtypeStruct((2 * m, n), jnp.bfloat16),
        in_specs=[pl.BlockSpec(memory_space=pltpu.VMEM)],
        out_specs=pl.BlockSpec(memory_space=pltpu.VMEM),
        scratch_shapes=[
            pltpu.SemaphoreType.DMA((C,)),
            pltpu.SemaphoreType.DMA((C,)),
            pltpu.SemaphoreType.DMA((C,)),
            pltpu.SemaphoreType.DMA((C,)),
            pltpu.SemaphoreType.DMA,
        ],
        compiler_params=pltpu.CompilerParams(collective_id=0),
    )(x)


# device time: 135931 ns/iter; 1.6833x vs baseline; 1.0948x over previous
import jax
import jax.numpy as jnp
from jax import lax
from jax.experimental import pallas as pl
from jax.experimental.pallas import tpu as pltpu

C = 16


def kernel(x):
    m, n = x.shape
    half = m // 2
    cs = half // C

    def body(x_ref, out_ref, stage, in_sems, y_send, y_recv, x_send, x_recv):
        my_x = lax.axis_index("x")
        my_y = lax.axis_index("y")
        my_z = lax.axis_index("z")
        y_peer = (my_x, 1 - my_y, my_z)
        x_peer = (1 - my_x, my_y, my_z)

        half_off = my_x * half
        oth_off = (1 - my_x) * half
        mine_off = my_y * m
        rem_off = (1 - my_y) * m

        in_dmas = []
        for c in range(C):
            dma = pltpu.make_async_copy(
                x_ref.at[pl.ds(half_off + c * cs, cs), :],
                stage.at[pl.ds(c * cs, cs), :],
                in_sems.at[c],
            )
            dma.start()
            in_dmas.append(dma)

        bar = pltpu.get_barrier_semaphore()
        for p in (y_peer, x_peer):
            pl.semaphore_signal(
                bar, inc=1, device_id=p, device_id_type=pl.DeviceIdType.MESH
            )
        pl.semaphore_wait(bar, 2)

        y_rdmas = []
        dma2 = []
        for c in range(C):
            in_dmas[c].wait()
            src_o = mine_off + half_off + c * cs
            out_ref[pl.ds(src_o, cs), :] = stage[pl.ds(c * cs, cs), :].astype(
                jnp.bfloat16
            )
            rdma = pltpu.make_async_remote_copy(
                src_ref=out_ref.at[pl.ds(src_o, cs), :],
                dst_ref=out_ref.at[pl.ds(src_o, cs), :],
                send_sem=y_send.at[c],
                recv_sem=y_recv.at[c],
                device_id=y_peer,
                device_id_type=pl.DeviceIdType.MESH,
            )
            rdma.start()
            y_rdmas.append(rdma)
            d2 = pltpu.make_async_copy(
                x_ref.at[pl.ds(oth_off + c * cs, cs), :],
                stage.at[pl.ds(c * cs, cs), :],
                in_sems.at[c],
            )
            d2.start()
            dma2.append(d2)

        x_rdmas = []
        for c in range(C):
            y_rdmas[c].wait_recv()
            f_off = rem_off + half_off + c * cs
            fwd = pltpu.make_async_remote_copy(
                src_ref=out_ref.at[pl.ds(f_off, cs), :],
                dst_ref=out_ref.at[pl.ds(f_off, cs), :],
                send_sem=x_send.at[c],
                recv_sem=x_recv.at[c],
                device_id=x_peer,
                device_id_type=pl.DeviceIdType.MESH,
            )
            fwd.start()
            dma2[c].wait()
            out_ref[pl.ds(mine_off + oth_off + c * cs, cs), :] = stage[
                pl.ds(c * cs, cs), :
            ].astype(jnp.bfloat16)
            x_rdmas.append(fwd)

        for c in range(C):
            x_rdmas[c].wait_recv()
            y_rdmas[c].wait_send()
            x_rdmas[c].wait_send()

    return pl.pallas_call(
        body,
        out_shape=jax.ShapeDtypeStruct((2 * m, n), jnp.bfloat16),
        in_specs=[pl.BlockSpec(memory_space=pl.ANY)],
        out_specs=pl.BlockSpec(memory_space=pltpu.VMEM),
        scratch_shapes=[
            pltpu.VMEM((half, n), jnp.float32),
            pltpu.SemaphoreType.DMA((C,)),
            pltpu.SemaphoreType.DMA((C,)),
            pltpu.SemaphoreType.DMA((C,)),
            pltpu.SemaphoreType.DMA((C,)),
            pltpu.SemaphoreType.DMA((C,)),
        ],
        compiler_params=pltpu.CompilerParams(
            collective_id=0, vmem_limit_bytes=60 * 1024 * 1024
        ),
    )(x)
